# Initial kernel scaffold; baseline (speedup 1.0000x reference)
import jax
import jax.numpy as jnp
from jax.experimental import pallas as pl


def kernel(hidden_states, kernel):
    raise NotImplementedError("write your pallas kernel here")


# speedup vs baseline: 4.5900x; 4.5900x over previous
import functools

import jax
import jax.numpy as jnp
from jax.experimental import pallas as pl
from jax.experimental.pallas import tpu as pltpu

N_EXP = 64
N_GRP = 8
EPG = 8
TOPK_G = 3
TOPK = 8
NEG = -1e30


def _gate_kernel(w_ref, h_ref, out_ref):
    logits = jax.lax.dot_general(
        w_ref[...], h_ref[...],
        (((1,), (1,)), ((), ())),
        preferred_element_type=jnp.float32,
    )
    T = logits.shape[1]

    gs = jnp.concatenate(
        [jnp.max(logits[g * EPG:(g + 1) * EPG], axis=0, keepdims=True)
         for g in range(N_GRP)], axis=0)

    gidx = jax.lax.broadcasted_iota(jnp.int32, (N_GRP, T), 0)
    cur = gs
    sel = jnp.zeros((N_GRP, T), jnp.bool_)
    for _ in range(TOPK_G):
        m = jnp.max(cur, axis=0, keepdims=True)
        cand = jnp.where(cur == m, gidx, N_GRP)
        amin = jnp.min(cand, axis=0, keepdims=True)
        pick = gidx == amin
        sel = jnp.logical_or(sel, pick)
        cur = jnp.where(pick, NEG, cur)

    sel64 = jnp.concatenate(
        [jnp.broadcast_to(sel[g:g + 1], (EPG, T)) for g in range(N_GRP)],
        axis=0)
    masked = jnp.where(sel64, logits, NEG)

    eidx = jax.lax.broadcasted_iota(jnp.int32, (N_EXP, T), 0)
    vals = []
    for _ in range(TOPK):
        m = jnp.max(masked, axis=0, keepdims=True)
        vals.append(m)
        cand = jnp.where(masked == m, eidx, N_EXP)
        amin = jnp.min(cand, axis=0, keepdims=True)
        masked = jnp.where(eidx == amin, NEG, masked)

    w = jnp.concatenate(vals, axis=0)
    e = jnp.exp(w - w[0:1])
    out = e / jnp.sum(e, axis=0, keepdims=True)
    out_ref[...] = out.T


def kernel(hidden_states, kernel):
    gate_w = kernel
    S, H = hidden_states.shape
    T = 512
    return pl.pallas_call(
        _gate_kernel,
        grid=(S // T,),
        in_specs=[
            pl.BlockSpec((N_EXP, H), lambda i: (0, 0)),
            pl.BlockSpec((T, H), lambda i: (i, 0)),
        ],
        out_specs=pl.BlockSpec((T, TOPK), lambda i: (i, 0)),
        out_shape=jax.ShapeDtypeStruct((S, TOPK), jnp.float32),
    )(gate_w, hidden_states)
